# baseline (speedup 1.0000x reference)
import functools

import jax
import jax.numpy as jnp
from jax import lax
from jax.experimental import pallas as pl
from jax.experimental.pallas import tpu as pltpu
from jax.experimental.pallas import tpu_sc as plsc

_CUT = (0, 20000, 40000, 200000, 267735)
_DS = (1024, 256, 64, 16)
_DP = 1024
_NTOK = 8192

_NC = 2
_NS = 16
_NW = _NC * _NS
_TPT = _NTOK // _NW
_CH = 32
_NCHUNK = _TPT // _CH


def _sc_gather(inp_flat, emb0, emb1, emb2, emb3):
    out_type = [jax.ShapeDtypeStruct((_NTOK, d), jnp.float32) for d in _DS]
    mesh = plsc.VectorSubcoreMesh(core_axis_name="c", subcore_axis_name="s")
    bufs = [pltpu.VMEM((_CH, d), jnp.float32) for d in _DS]
    scratch_types = (
        [pltpu.VMEM((_TPT,), jnp.int32) for _ in range(3)]
        + bufs + bufs
        + [pltpu.SemaphoreType.DMA] * 6
    )

    @functools.partial(
        pl.kernel, mesh=mesh, out_type=out_type, scratch_types=scratch_types
    )
    def k(inp_hbm, e0, e1, e2, e3, x0, x1, x2, x3,
          inp_v, i0, i1,
          b0a, b1a, b2a, b3a, b0b, b1b, b2b, b3b,
          gsa, gsb, rsa, rsb, wsa, wsb):
        wid = lax.axis_index("s") * _NC + lax.axis_index("c")
        base = wid * _TPT
        pltpu.sync_copy(inp_hbm.at[pl.ds(base, _TPT)], inp_v)
        for j in range(_TPT // 16):
            sl = pl.ds(j * 16, 16)
            v = inp_v[sl]
            i0[sl] = jnp.clip(v, 0, _CUT[1] - 1)
            i1[sl] = jnp.clip(v - _CUT[1], 0, _CUT[2] - _CUT[1] - 1)

        tabs = (e0, e1, e2, e3)
        xs = (x0, x1, x2, x3)
        ilists = (i0, i1)
        sets = ((b0a, b1a, b2a, b3a, gsa, rsa, wsa),
                (b0b, b1b, b2b, b3b, gsb, rsb, wsb))

        def rows(ch, p, fire):
            s = sets[p]
            for j in range(_CH // 16):
                v = inp_v[pl.ds(ch * _CH + j * 16, 16)]
                for l in range(16):
                    t = v[l]
                    for c in (2, 3):
                        @pl.when((t >= _CUT[c]) & (t < _CUT[c + 1]))
                        def _(c=c, t=t, l=l, j=j, s=s):
                            cp = pltpu.make_async_copy(
                                tabs[c].at[pl.ds(t - _CUT[c], 1)],
                                s[c].at[pl.ds(j * 16 + l, 1)], s[5])
                            if fire:
                                cp.start()
                            else:
                                cp.wait()

        def fire_g(ch, p):
            s = sets[p]
            cps = [
                pltpu.async_copy(
                    tabs[c].at[ilists[c].at[pl.ds(ch * _CH, _CH)]],
                    s[c], s[4])
                for c in range(2)
            ]
            rows(ch, p, True)
            return cps

        def wait_g(ch, p, cps):
            for cp in cps:
                cp.wait()
            rows(ch, p, False)

        def fire_w(ch, p):
            s = sets[p]
            return [
                pltpu.async_copy(
                    s[c], xs[c].at[pl.ds(base + ch * _CH, _CH)], s[6])
                for c in range(4)
            ]

        def pair(it):
            ch = it * 2
            ga = fire_g(ch, 0)
            gb = fire_g(ch + 1, 1)
            wait_g(ch, 0, ga)
            wa = fire_w(ch, 0)
            wait_g(ch + 1, 1, gb)
            wb = fire_w(ch + 1, 1)
            for cp in wa + wb:
                cp.wait()

        pl.loop(0, _NCHUNK // 2)(pair)

    return k(inp_flat, emb0, emb1, emb2, emb3)


def _tc_matmul(inp2d, x0, x1, x2, x3, p0, p1, p2, p3):
    bm = 256
    grid = (_NTOK // bm,)

    def body(inp_ref, x0r, x1r, x2r, x3r, p0r, p1r, p2r, p3r, o_ref):
        iv = inp_ref[...]
        acc = jnp.zeros((bm, _DP), jnp.float32)
        for c, (xr, pr) in enumerate(
                ((x0r, p0r), (x1r, p1r), (x2r, p2r), (x3r, p3r))):
            m = (iv >= _CUT[c]) & (iv < _CUT[c + 1])
            xc = jnp.where(m, xr[...], 0.0).astype(jnp.bfloat16)
            acc = acc + jnp.dot(xc, pr[...],
                                preferred_element_type=jnp.float32)
        o_ref[...] = acc * (_DP ** 0.5)

    in_specs = (
        [pl.BlockSpec((bm, 1), lambda i: (i, 0))]
        + [pl.BlockSpec((bm, d), lambda i: (i, 0)) for d in _DS]
        + [pl.BlockSpec((d, _DP), lambda i: (0, 0)) for d in _DS]
    )
    return pl.pallas_call(
        body,
        grid=grid,
        in_specs=in_specs,
        out_specs=pl.BlockSpec((bm, _DP), lambda i: (i, 0)),
        out_shape=jax.ShapeDtypeStruct((_NTOK, _DP), jnp.float32),
    )(inp2d, x0, x1, x2, x3, p0, p1, p2, p3)


@jax.jit
def kernel(inp, emb0, emb1, emb2, emb3, proj0, proj1, proj2, proj3):
    inp_flat = inp.reshape(-1)
    xs = _sc_gather(inp_flat, emb0, emb1, emb2, emb3)
    ps = [p.astype(jnp.bfloat16) for p in (proj0, proj1, proj2, proj3)]
    out = _tc_matmul(inp_flat.reshape(-1, 1), *xs, *ps)
    return out.reshape(inp.shape + (_DP,))


# speedup vs baseline: 2.3183x; 2.3183x over previous
import functools

import jax
import jax.numpy as jnp
from jax import lax
from jax.experimental import pallas as pl
from jax.experimental.pallas import tpu as pltpu
from jax.experimental.pallas import tpu_sc as plsc

_CUT = (0, 20000, 40000, 200000, 267735)
_DS = (1024, 256, 64, 16)
_DP = 1024
_NTOK = 8192

_NC = 2
_NS = 16
_NW = _NC * _NS
_TPT = _NTOK // _NW
_CH = 32
_NCHUNK = _TPT // _CH


def _sc_gather(inp_flat, emb0, emb1, emb2, emb3):
    out_type = [jax.ShapeDtypeStruct((_NTOK, d), jnp.float32) for d in _DS]
    mesh = plsc.VectorSubcoreMesh(core_axis_name="c", subcore_axis_name="s")
    bufs = [pltpu.VMEM((_CH, d), jnp.float32) for d in (64, 16)]
    scratch_types = (
        [pltpu.VMEM((_TPT,), jnp.int32)]
        + bufs + bufs
        + [pltpu.SemaphoreType.DMA] * 4
    )

    @functools.partial(
        pl.kernel, mesh=mesh, out_type=out_type, scratch_types=scratch_types
    )
    def k(inp_hbm, e0, e1, e2, e3, x0, x1, x2, x3,
          inp_v, b2a, b3a, b2b, b3b, rsa, rsb, wsa, wsb):
        wid = lax.axis_index("s") * _NC + lax.axis_index("c")
        base = wid * _TPT
        pltpu.sync_copy(inp_hbm.at[pl.ds(base, _TPT)], inp_v)

        tabs = (e0, e1, e2, e3)
        xs = (x0, x1, x2, x3)
        sets = ((b2a, b3a, rsa, wsa), (b2b, b3b, rsb, wsb))

        def rows(ch, p, fire):
            s = sets[p]
            for j in range(_CH // 16):
                v = inp_v[pl.ds(ch * _CH + j * 16, 16)]
                for l in range(16):
                    t = v[l]
                    tok = base + ch * _CH + j * 16 + l
                    for c in range(4):
                        @pl.when((t >= _CUT[c]) & (t < _CUT[c + 1]))
                        def _(c=c, t=t, l=l, j=j, s=s, tok=tok):
                            if c < 2:
                                dst = xs[c].at[pl.ds(tok, 1)]
                            else:
                                dst = s[c - 2].at[pl.ds(j * 16 + l, 1)]
                            cp = pltpu.make_async_copy(
                                tabs[c].at[pl.ds(t - _CUT[c], 1)],
                                dst, s[2])
                            if fire:
                                cp.start()
                            else:
                                cp.wait()

        def fire_w(ch, p):
            s = sets[p]
            return [
                pltpu.async_copy(
                    s[c], xs[c + 2].at[pl.ds(base + ch * _CH, _CH)], s[3])
                for c in range(2)
            ]

        def pair(it):
            ch = it * 2
            rows(ch, 0, True)
            rows(ch + 1, 1, True)
            rows(ch, 0, False)
            wa = fire_w(ch, 0)
            rows(ch + 1, 1, False)
            wb = fire_w(ch + 1, 1)
            for cp in wa + wb:
                cp.wait()

        pl.loop(0, _NCHUNK // 2)(pair)

    return k(inp_flat, emb0, emb1, emb2, emb3)


def _tc_matmul(inp2d, x0, x1, x2, x3, p0, p1, p2, p3):
    bm = 256
    grid = (_NTOK // bm,)

    def body(inp_ref, x0r, x1r, x2r, x3r, p0r, p1r, p2r, p3r, o_ref):
        iv = inp_ref[...]
        acc = jnp.zeros((bm, _DP), jnp.float32)
        for c, (xr, pr) in enumerate(
                ((x0r, p0r), (x1r, p1r), (x2r, p2r), (x3r, p3r))):
            m = (iv >= _CUT[c]) & (iv < _CUT[c + 1])
            xc = jnp.where(m, xr[...], 0.0).astype(jnp.bfloat16)
            acc = acc + jnp.dot(xc, pr[...],
                                preferred_element_type=jnp.float32)
        o_ref[...] = acc * (_DP ** 0.5)

    in_specs = (
        [pl.BlockSpec((bm, 1), lambda i: (i, 0))]
        + [pl.BlockSpec((bm, d), lambda i: (i, 0)) for d in _DS]
        + [pl.BlockSpec((d, _DP), lambda i: (0, 0)) for d in _DS]
    )
    return pl.pallas_call(
        body,
        grid=grid,
        in_specs=in_specs,
        out_specs=pl.BlockSpec((bm, _DP), lambda i: (i, 0)),
        out_shape=jax.ShapeDtypeStruct((_NTOK, _DP), jnp.float32),
    )(inp2d, x0, x1, x2, x3, p0, p1, p2, p3)


@jax.jit
def kernel(inp, emb0, emb1, emb2, emb3, proj0, proj1, proj2, proj3):
    inp_flat = inp.reshape(-1)
    xs = _sc_gather(inp_flat, emb0, emb1, emb2, emb3)
    ps = [p.astype(jnp.bfloat16) for p in (proj0, proj1, proj2, proj3)]
    out = _tc_matmul(inp_flat.reshape(-1, 1), *xs, *ps)
    return out.reshape(inp.shape + (_DP,))


# speedup vs baseline: 2.4032x; 1.0366x over previous
import functools

import jax
import jax.numpy as jnp
from jax import lax
from jax.experimental import pallas as pl
from jax.experimental.pallas import tpu as pltpu
from jax.experimental.pallas import tpu_sc as plsc

_CUT = (0, 20000, 40000, 200000, 267735)
_DS = (1024, 256, 64, 16)
_DP = 1024
_NTOK = 8192

_NC = 2
_NS = 16
_NW = _NC * _NS
_TPT = _NTOK // _NW
_CH = 32
_NCHUNK = _TPT // _CH


def _sc_gather(inp_flat, emb0, emb1, emb2, emb3):
    out_type = [jax.ShapeDtypeStruct((_NTOK, d), jnp.float32) for d in _DS]
    mesh = plsc.VectorSubcoreMesh(core_axis_name="c", subcore_axis_name="s")
    bufs = [pltpu.VMEM((_CH, d), jnp.float32) for d in (64, 16)]
    scratch_types = (
        [pltpu.VMEM((_TPT,), jnp.int32)]
        + bufs + bufs
        + [pltpu.SemaphoreType.DMA] * 4
    )

    @functools.partial(
        pl.kernel, mesh=mesh, out_type=out_type, scratch_types=scratch_types
    )
    def k(inp_hbm, e0, e1, e2, e3, x0, x1, x2, x3,
          inp_v, b2a, b3a, b2b, b3b, rsa, rsb, wsa, wsb):
        wid = lax.axis_index("s") * _NC + lax.axis_index("c")
        base = wid * _TPT
        pltpu.sync_copy(inp_hbm.at[pl.ds(base, _TPT)], inp_v)

        tabs = (e0, e1, e2, e3)
        xs = (x0, x1, x2, x3)
        sets = ((b2a, b3a, rsa, wsa), (b2b, b3b, rsb, wsb))

        def rows(ch, p, fire):
            s = sets[p]
            for j in range(_CH // 16):
                v = inp_v[pl.ds(ch * _CH + j * 16, 16)]
                for l in range(16):
                    t = v[l]
                    tok = base + ch * _CH + j * 16 + l
                    for c in range(4):
                        @pl.when((t >= _CUT[c]) & (t < _CUT[c + 1]))
                        def _(c=c, t=t, l=l, j=j, s=s, tok=tok):
                            if c < 2:
                                dst = xs[c].at[pl.ds(tok, 1)]
                            else:
                                dst = s[c - 2].at[pl.ds(j * 16 + l, 1)]
                            cp = pltpu.make_async_copy(
                                tabs[c].at[pl.ds(t - _CUT[c], 1)],
                                dst, s[2])
                            if fire:
                                cp.start()
                            else:
                                cp.wait()

        def fire_w(ch, p):
            s = sets[p]
            return [
                pltpu.async_copy(
                    s[c], xs[c + 2].at[pl.ds(base + ch * _CH, _CH)], s[3])
                for c in range(2)
            ]

        def pair(it):
            ch = it * 2
            rows(ch, 0, True)
            rows(ch + 1, 1, True)
            rows(ch, 0, False)
            wa = fire_w(ch, 0)
            rows(ch + 1, 1, False)
            wb = fire_w(ch + 1, 1)
            for cp in wa + wb:
                cp.wait()

        pl.loop(0, _NCHUNK // 2)(pair)

    return k(inp_flat, emb0, emb1, emb2, emb3)


def _tc_matmul(inp2d, x0, x1, x2, x3, p0, p1, p2, p3):
    bm = 512
    grid = (_NTOK // bm,)

    def body(inp_ref, x0r, x1r, x2r, x3r, p0r, p1r, p2r, p3r, o_ref):
        iv = inp_ref[...]
        acc = jnp.zeros((bm, _DP), jnp.float32)
        for c, (xr, pr) in enumerate(
                ((x0r, p0r), (x1r, p1r), (x2r, p2r), (x3r, p3r))):
            m = (iv >= _CUT[c]) & (iv < _CUT[c + 1])
            xc = jnp.where(m, xr[...], 0.0).astype(jnp.bfloat16)
            acc = acc + jnp.dot(xc, pr[...],
                                preferred_element_type=jnp.float32)
        o_ref[...] = acc * (_DP ** 0.5)

    in_specs = (
        [pl.BlockSpec((bm, 1), lambda i: (i, 0))]
        + [pl.BlockSpec((bm, d), lambda i: (i, 0)) for d in _DS]
        + [pl.BlockSpec((d, _DP), lambda i: (0, 0)) for d in _DS]
    )
    return pl.pallas_call(
        body,
        grid=grid,
        in_specs=in_specs,
        out_specs=pl.BlockSpec((bm, _DP), lambda i: (i, 0)),
        out_shape=jax.ShapeDtypeStruct((_NTOK, _DP), jnp.float32),
    )(inp2d, x0, x1, x2, x3, p0, p1, p2, p3)


@jax.jit
def kernel(inp, emb0, emb1, emb2, emb3, proj0, proj1, proj2, proj3):
    inp_flat = inp.reshape(-1)
    xs = _sc_gather(inp_flat, emb0, emb1, emb2, emb3)
    ps = [p.astype(jnp.bfloat16) for p in (proj0, proj1, proj2, proj3)]
    out = _tc_matmul(inp_flat.reshape(-1, 1), *xs, *ps)
    return out.reshape(inp.shape + (_DP,))


# speedup vs baseline: 2.4437x; 1.0169x over previous
import functools

import jax
import jax.numpy as jnp
from jax import lax
from jax.experimental import pallas as pl
from jax.experimental.pallas import tpu as pltpu
from jax.experimental.pallas import tpu_sc as plsc

_CUT = (0, 20000, 40000, 200000, 267735)
_DS = (1024, 256, 64, 16)
_DP = 1024
_NTOK = 8192

_NC = 2
_NS = 16
_NW = _NC * _NS
_TPT = _NTOK // _NW
_CH = 32
_NCHUNK = _TPT // _CH


def _sc_gather(inp_flat, emb0, emb1, emb2, emb3):
    out_type = [jax.ShapeDtypeStruct((_NTOK, d), jnp.float32) for d in _DS]
    mesh = plsc.VectorSubcoreMesh(core_axis_name="c", subcore_axis_name="s")
    bufs = [pltpu.VMEM((_CH, d), jnp.float32) for d in (64, 16)]
    scratch_types = (
        [pltpu.VMEM((_TPT,), jnp.int32)]
        + bufs + bufs
        + [pltpu.SemaphoreType.DMA] * 4
    )

    @functools.partial(
        pl.kernel, mesh=mesh, out_type=out_type, scratch_types=scratch_types
    )
    def k(inp_hbm, e0, e1, e2, e3, x0, x1, x2, x3,
          inp_v, b2a, b3a, b2b, b3b, rsa, rsb, wsa, wsb):
        wid = lax.axis_index("s") * _NC + lax.axis_index("c")
        base = wid * _TPT
        pltpu.sync_copy(inp_hbm.at[pl.ds(base, _TPT)], inp_v)

        tabs = (e0, e1, e2, e3)
        xs = (x0, x1, x2, x3)
        sets = ((b2a, b3a, rsa, wsa), (b2b, b3b, rsb, wsb))

        def rows(ch, p, fire):
            s = sets[p]
            for j in range(_CH // 16):
                v = inp_v[pl.ds(ch * _CH + j * 16, 16)]
                for l in range(16):
                    t = v[l]
                    tok = base + ch * _CH + j * 16 + l
                    for c in range(4):
                        @pl.when((t >= _CUT[c]) & (t < _CUT[c + 1]))
                        def _(c=c, t=t, l=l, j=j, s=s, tok=tok):
                            if c < 2:
                                dst = xs[c].at[pl.ds(tok, 1)]
                            else:
                                dst = s[c - 2].at[pl.ds(j * 16 + l, 1)]
                            cp = pltpu.make_async_copy(
                                tabs[c].at[pl.ds(t - _CUT[c], 1)],
                                dst, s[2])
                            if fire:
                                cp.start()
                            else:
                                cp.wait()

        def fire_w(ch, p):
            s = sets[p]
            return [
                pltpu.async_copy(
                    s[c], xs[c + 2].at[pl.ds(base + ch * _CH, _CH)], s[3])
                for c in range(2)
            ]

        def pair(it):
            ch = it * 2
            rows(ch, 0, True)
            rows(ch + 1, 1, True)
            rows(ch, 0, False)
            wa = fire_w(ch, 0)
            rows(ch + 1, 1, False)
            wb = fire_w(ch + 1, 1)
            for cp in wa + wb:
                cp.wait()

        pl.loop(0, _NCHUNK // 2)(pair)

    return k(inp_flat, emb0, emb1, emb2, emb3)


def _tc_matmul(inp2d, x0, x1, x2, x3, p0, p1, p2, p3):
    bm = 1024
    grid = (_NTOK // bm,)

    def body(inp_ref, x0r, x1r, x2r, x3r, p0r, p1r, p2r, p3r, o_ref):
        iv = inp_ref[...]
        acc = jnp.zeros((bm, _DP), jnp.float32)
        for c, (xr, pr) in enumerate(
                ((x0r, p0r), (x1r, p1r), (x2r, p2r), (x3r, p3r))):
            m = (iv >= _CUT[c]) & (iv < _CUT[c + 1])
            xc = jnp.where(m, xr[...], 0.0).astype(jnp.bfloat16)
            acc = acc + jnp.dot(xc, pr[...],
                                preferred_element_type=jnp.float32)
        o_ref[...] = acc * (_DP ** 0.5)

    in_specs = (
        [pl.BlockSpec((bm, 1), lambda i: (i, 0))]
        + [pl.BlockSpec((bm, d), lambda i: (i, 0)) for d in _DS]
        + [pl.BlockSpec((d, _DP), lambda i: (0, 0)) for d in _DS]
    )
    return pl.pallas_call(
        body,
        grid=grid,
        in_specs=in_specs,
        out_specs=pl.BlockSpec((bm, _DP), lambda i: (i, 0)),
        out_shape=jax.ShapeDtypeStruct((_NTOK, _DP), jnp.float32),
    )(inp2d, x0, x1, x2, x3, p0, p1, p2, p3)


@jax.jit
def kernel(inp, emb0, emb1, emb2, emb3, proj0, proj1, proj2, proj3):
    inp_flat = inp.reshape(-1)
    xs = _sc_gather(inp_flat, emb0, emb1, emb2, emb3)
    ps = [p.astype(jnp.bfloat16) for p in (proj0, proj1, proj2, proj3)]
    out = _tc_matmul(inp_flat.reshape(-1, 1), *xs, *ps)
    return out.reshape(inp.shape + (_DP,))
